# baseline (speedup 1.0000x reference)
import functools

import jax
import jax.numpy as jnp
from jax import lax
from jax.experimental import pallas as pl
from jax.experimental.pallas import tpu as pltpu
from jax.experimental.pallas import tpu_sc as plsc

BN = 512
K = 8192
CHAIN_BOUNDS = (0, 2736, 5472, 8192)


def _bf16_round(v):
    return v.astype(jnp.bfloat16).astype(jnp.float32)


def _argmin_body(x_ref, cb_ref, xsq_ref, csq_ref, idx_ref, loss_ref):
    xb = x_ref[...]
    cb = cb_ref[...]
    x_sq = xsq_ref[0]
    c_sq = csq_ref[0]
    mm = lax.dot_general(xb, cb, (((1,), (1,)), ((), ())),
                         preferred_element_type=jnp.float32)
    dist = (x_sq - 2.0 * mm) + c_sq
    j = lax.broadcasted_iota(jnp.int32, (BN, K), 1)

    def chain(lo, hi):
        mask = (j >= lo) & (j < hi)
        dc = jnp.where(mask, dist, jnp.inf)
        m = jnp.min(dc, axis=1, keepdims=True)
        i = jnp.min(jnp.where(dc == m, j, K), axis=1,
                    keepdims=True)
        return m, i

    m0, i0 = chain(CHAIN_BOUNDS[0], CHAIN_BOUNDS[1])
    m1, i1 = chain(CHAIN_BOUNDS[1], CHAIN_BOUNDS[2])
    m2, i2 = chain(CHAIN_BOUNDS[2], CHAIN_BOUNDS[3])

    acc_v, acc_i, sel_m = _bf16_round(m0), i0, m0
    for m, i in ((m1, i1), (m2, i2)):
        win = (m < acc_v) | ((m == acc_v) & (i < acc_i))
        acc_i = jnp.where(win, i, acc_i)
        sel_m = jnp.where(win, m, sel_m)
        acc_v = jnp.where(win, _bf16_round(m), acc_v)

    idx_ref[0, ...] = acc_i.reshape(1, BN)
    loss_ref[...] = jnp.full((1, 1, 1), jnp.sum(sel_m), jnp.float32)


def _dist_argmin(flat, codebook, x_sq, c_sq):
    n, c = flat.shape
    nb = n // BN
    return pl.pallas_call(
        _argmin_body,
        grid=(nb,),
        in_specs=[
            pl.BlockSpec((BN, c), lambda i: (i, 0)),
            pl.BlockSpec((K, c), lambda i: (0, 0)),
            pl.BlockSpec((1, BN, 1), lambda i: (i, 0, 0)),
            pl.BlockSpec((1, 1, K), lambda i: (0, 0, 0)),
        ],
        out_specs=[
            pl.BlockSpec((1, 1, BN), lambda i: (i, 0, 0)),
            pl.BlockSpec((1, 1, 1), lambda i: (i, 0, 0)),
        ],
        out_shape=[
            jax.ShapeDtypeStruct((nb, 1, BN), jnp.int32),
            jax.ShapeDtypeStruct((nb, 1, 1), jnp.float32),
        ],
        compiler_params=pltpu.CompilerParams(
            dimension_semantics=("parallel",)),
    )(flat, codebook, x_sq.reshape(nb, BN, 1), c_sq.reshape(1, 1, K))


def _gather_rows_sc(table, idx_flat):
    n = idx_flat.shape[0]
    d = table.shape[1]
    info = plsc.get_sparse_core_info()
    nw = info.num_cores * info.num_subcores
    b_per_w = n // nw
    ch = 128
    nch = b_per_w // ch
    mesh = plsc.VectorSubcoreMesh(core_axis_name="c", subcore_axis_name="s")

    @functools.partial(
        pl.kernel, mesh=mesh,
        out_type=jax.ShapeDtypeStruct((n, d), jnp.float32),
        scratch_types=[
            pltpu.VMEM((ch,), jnp.int32),
            pltpu.VMEM((ch, d), jnp.float32),
            pltpu.SemaphoreType.DMA,
        ],
    )
    def gather_kernel(table_hbm, idx_hbm, out_hbm, idx_v, rows_v, sem):
        wid = lax.axis_index("s") * info.num_cores + lax.axis_index("c")
        base = wid * b_per_w
        for cchunk in range(nch):
            off = base + cchunk * ch
            pltpu.sync_copy(idx_hbm.at[pl.ds(off, ch)], idx_v)
            pltpu.async_copy(table_hbm.at[idx_v], rows_v, sem).wait()
            pltpu.sync_copy(rows_v, out_hbm.at[pl.ds(off, ch)])

    return gather_kernel(table, idx_flat)


def kernel(x, codebook):
    b, t, c = x.shape
    n = b * t
    flat = x.reshape(n, c)
    x_sq = jnp.sum(x * x, axis=-1).reshape(n, 1)
    c_sq = jnp.sum(codebook * codebook, axis=1)
    idx3, loss3 = _dist_argmin(flat, codebook, x_sq, c_sq)
    idx_flat = idx3.reshape(n)
    qe_flat = _gather_rows_sc(codebook, idx_flat)
    qe = qe_flat.reshape(x.shape)
    commit_loss = loss3.sum() / jnp.float32(n * c)
    indices = idx_flat.reshape(b, t)
    return qe, commit_loss, indices


# speedup vs baseline: 1.1866x; 1.1866x over previous
import functools

import jax
import jax.numpy as jnp
from jax import lax
from jax.experimental import pallas as pl
from jax.experimental.pallas import tpu as pltpu
from jax.experimental.pallas import tpu_sc as plsc

BN = 512
K = 8192
CHAIN_BOUNDS = (0, 2736, 5472, 8192)


def _bf16_round(v):
    return v.astype(jnp.bfloat16).astype(jnp.float32)


def _argmin_body(x_ref, cb_ref, xsq_ref, csq_ref, idx_ref, loss_ref):
    xb = x_ref[...]
    cb = cb_ref[...]
    x_sq = xsq_ref[0]
    c_sq = csq_ref[0]
    mm = lax.dot_general(xb, cb, (((1,), (1,)), ((), ())),
                         preferred_element_type=jnp.float32)
    dist = (x_sq - 2.0 * mm) + c_sq
    j = lax.broadcasted_iota(jnp.int32, (BN, K), 1)
    jf = j.astype(jnp.float32)

    def chain(lo, hi):
        mask = (j >= lo) & (j < hi)
        dc = jnp.where(mask, dist, jnp.inf)
        m = jnp.min(dc, axis=1, keepdims=True)
        i_f = jnp.min(jnp.where(dc == m, jf, jnp.float32(K)), axis=1,
                      keepdims=True)
        return m, i_f.astype(jnp.int32)

    m0, i0 = chain(CHAIN_BOUNDS[0], CHAIN_BOUNDS[1])
    m1, i1 = chain(CHAIN_BOUNDS[1], CHAIN_BOUNDS[2])
    m2, i2 = chain(CHAIN_BOUNDS[2], CHAIN_BOUNDS[3])

    acc_v, acc_i, sel_m = _bf16_round(m0), i0, m0
    for m, i in ((m1, i1), (m2, i2)):
        win = (m < acc_v) | ((m == acc_v) & (i < acc_i))
        acc_i = jnp.where(win, i, acc_i)
        sel_m = jnp.where(win, m, sel_m)
        acc_v = jnp.where(win, _bf16_round(m), acc_v)

    idx_ref[0, ...] = acc_i.reshape(1, BN)
    loss_ref[...] = jnp.full((1, 1, 1), jnp.sum(sel_m), jnp.float32)


def _dist_argmin(flat, codebook, x_sq, c_sq):
    n, c = flat.shape
    nb = n // BN
    return pl.pallas_call(
        _argmin_body,
        grid=(nb,),
        in_specs=[
            pl.BlockSpec((BN, c), lambda i: (i, 0)),
            pl.BlockSpec((K, c), lambda i: (0, 0)),
            pl.BlockSpec((1, BN, 1), lambda i: (i, 0, 0)),
            pl.BlockSpec((1, 1, K), lambda i: (0, 0, 0)),
        ],
        out_specs=[
            pl.BlockSpec((1, 1, BN), lambda i: (i, 0, 0)),
            pl.BlockSpec((1, 1, 1), lambda i: (i, 0, 0)),
        ],
        out_shape=[
            jax.ShapeDtypeStruct((nb, 1, BN), jnp.int32),
            jax.ShapeDtypeStruct((nb, 1, 1), jnp.float32),
        ],
        compiler_params=pltpu.CompilerParams(
            dimension_semantics=("parallel",)),
    )(flat, codebook, x_sq.reshape(nb, BN, 1), c_sq.reshape(1, 1, K))


def _gather_rows_sc(table, idx_flat):
    n = idx_flat.shape[0]
    d = table.shape[1]
    info = plsc.get_sparse_core_info()
    nw = info.num_cores * info.num_subcores
    b_per_w = n // nw
    ch = 128
    nch = b_per_w // ch
    mesh = plsc.VectorSubcoreMesh(core_axis_name="c", subcore_axis_name="s")

    @functools.partial(
        pl.kernel, mesh=mesh,
        out_type=jax.ShapeDtypeStruct((n, d), jnp.float32),
        scratch_types=[
            pltpu.VMEM((ch,), jnp.int32),
            pltpu.VMEM((ch, d), jnp.float32),
            pltpu.SemaphoreType.DMA,
        ],
    )
    def gather_kernel(table_hbm, idx_hbm, out_hbm, idx_v, rows_v, sem):
        wid = lax.axis_index("s") * info.num_cores + lax.axis_index("c")
        base = wid * b_per_w
        for cchunk in range(nch):
            off = base + cchunk * ch
            pltpu.sync_copy(idx_hbm.at[pl.ds(off, ch)], idx_v)
            pltpu.async_copy(table_hbm.at[idx_v], rows_v, sem).wait()
            pltpu.sync_copy(rows_v, out_hbm.at[pl.ds(off, ch)])

    return gather_kernel(table, idx_flat)


def kernel(x, codebook):
    b, t, c = x.shape
    n = b * t
    flat = x.reshape(n, c)
    x_sq = jnp.sum(x * x, axis=-1).reshape(n, 1)
    c_sq = jnp.sum(codebook * codebook, axis=1)
    idx3, loss3 = _dist_argmin(flat, codebook, x_sq, c_sq)
    idx_flat = idx3.reshape(n)
    qe_flat = _gather_rows_sc(codebook, idx_flat)
    qe = qe_flat.reshape(x.shape)
    commit_loss = loss3.sum() / jnp.float32(n * c)
    indices = idx_flat.reshape(b, t)
    return qe, commit_loss, indices


# speedup vs baseline: 1.3436x; 1.1323x over previous
import functools

import jax
import jax.numpy as jnp
from jax import lax
from jax.experimental import pallas as pl
from jax.experimental.pallas import tpu as pltpu
from jax.experimental.pallas import tpu_sc as plsc

BN = 512
K = 8192
CHAIN_BOUNDS = (0, 2736, 5472, 8192)


def _bf16_round(v):
    return v.astype(jnp.bfloat16).astype(jnp.float32)


def _argmin_body(x_ref, cb_ref, xsq_ref, csq_ref, idx_ref, loss_ref):
    xb = x_ref[...]
    cb = cb_ref[...]
    x_sq = xsq_ref[0]
    c_sq = csq_ref[0]
    mm = lax.dot_general(xb, cb, (((1,), (1,)), ((), ())),
                         preferred_element_type=jnp.float32)
    dist = (x_sq - 2.0 * mm) + c_sq
    j = lax.broadcasted_iota(jnp.int32, (BN, K), 1)
    jf = j.astype(jnp.float32)

    def chain(lo, hi):
        mask = (j >= lo) & (j < hi)
        dc = jnp.where(mask, dist, jnp.inf)
        m = jnp.min(dc, axis=1, keepdims=True)
        return m, dc

    m0, dc0 = chain(CHAIN_BOUNDS[0], CHAIN_BOUNDS[1])
    m1, dc1 = chain(CHAIN_BOUNDS[1], CHAIN_BOUNDS[2])
    m2, dc2 = chain(CHAIN_BOUNDS[2], CHAIN_BOUNDS[3])

    a0 = _bf16_round(m0)
    win1 = m1 < a0
    a1 = jnp.where(win1, _bf16_round(m1), a0)
    win2 = m2 < a1
    sel_m = jnp.where(win2, m2, jnp.where(win1, m1, m0))
    dc_sel = jnp.where(win2, dc2, jnp.where(win1, dc1, dc0))
    i_f = jnp.min(jnp.where(dc_sel == sel_m, jf, jnp.float32(K)), axis=1,
                  keepdims=True)

    idx_ref[0, ...] = i_f.astype(jnp.int32).reshape(1, BN)
    loss_ref[...] = jnp.full((1, 1, 1), jnp.sum(sel_m), jnp.float32)


def _dist_argmin(flat, codebook, x_sq, c_sq):
    n, c = flat.shape
    nb = n // BN
    return pl.pallas_call(
        _argmin_body,
        grid=(nb,),
        in_specs=[
            pl.BlockSpec((BN, c), lambda i: (i, 0)),
            pl.BlockSpec((K, c), lambda i: (0, 0)),
            pl.BlockSpec((1, BN, 1), lambda i: (i, 0, 0)),
            pl.BlockSpec((1, 1, K), lambda i: (0, 0, 0)),
        ],
        out_specs=[
            pl.BlockSpec((1, 1, BN), lambda i: (i, 0, 0)),
            pl.BlockSpec((1, 1, 1), lambda i: (i, 0, 0)),
        ],
        out_shape=[
            jax.ShapeDtypeStruct((nb, 1, BN), jnp.int32),
            jax.ShapeDtypeStruct((nb, 1, 1), jnp.float32),
        ],
        compiler_params=pltpu.CompilerParams(
            dimension_semantics=("parallel",)),
    )(flat, codebook, x_sq.reshape(nb, BN, 1), c_sq.reshape(1, 1, K))


def _gather_rows_sc(table, idx_flat):
    n = idx_flat.shape[0]
    d = table.shape[1]
    info = plsc.get_sparse_core_info()
    nw = info.num_cores * info.num_subcores
    b_per_w = n // nw
    ch = 128
    nch = b_per_w // ch
    mesh = plsc.VectorSubcoreMesh(core_axis_name="c", subcore_axis_name="s")

    @functools.partial(
        pl.kernel, mesh=mesh,
        out_type=jax.ShapeDtypeStruct((n, d), jnp.float32),
        scratch_types=[
            pltpu.VMEM((ch,), jnp.int32),
            pltpu.VMEM((ch, d), jnp.float32),
            pltpu.SemaphoreType.DMA,
        ],
    )
    def gather_kernel(table_hbm, idx_hbm, out_hbm, idx_v, rows_v, sem):
        wid = lax.axis_index("s") * info.num_cores + lax.axis_index("c")
        base = wid * b_per_w
        for cchunk in range(nch):
            off = base + cchunk * ch
            pltpu.sync_copy(idx_hbm.at[pl.ds(off, ch)], idx_v)
            pltpu.async_copy(table_hbm.at[idx_v], rows_v, sem).wait()
            pltpu.sync_copy(rows_v, out_hbm.at[pl.ds(off, ch)])

    return gather_kernel(table, idx_flat)


def kernel(x, codebook):
    b, t, c = x.shape
    n = b * t
    flat = x.reshape(n, c)
    x_sq = jnp.sum(x * x, axis=-1).reshape(n, 1)
    c_sq = jnp.sum(codebook * codebook, axis=1)
    idx3, loss3 = _dist_argmin(flat, codebook, x_sq, c_sq)
    idx_flat = idx3.reshape(n)
    qe_flat = _gather_rows_sc(codebook, idx_flat)
    qe = qe_flat.reshape(x.shape)
    commit_loss = loss3.sum() / jnp.float32(n * c)
    indices = idx_flat.reshape(b, t)
    return qe, commit_loss, indices
